# Initial kernel scaffold; baseline (speedup 1.0000x reference)
import jax
import jax.numpy as jnp
from jax.experimental import pallas as pl


def kernel(user_emb, item_emb, creator_feat, item_feat, Wc, bc, Wi, bi, adj_values, layer_weights, adj_indices):
    raise NotImplementedError("write your pallas kernel here")


# speedup vs baseline: 4.1776x; 4.1776x over previous
import functools

import jax
import jax.numpy as jnp
from jax import lax
from jax.experimental import pallas as pl
from jax.experimental.pallas import tpu as pltpu
from jax.experimental.pallas import tpu_sc as plsc

N_USERS = 5000
M_ITEMS = 5000
N_NODES = N_USERS + M_ITEMS
N_EDGES = 320000
D = 128
N_LAYERS = 3

NC = 2
NS = 16
NW = NC * NS

CHUNK = 128
N_CHUNKS = 80
E_PER_TILE = CHUNK * N_CHUNKS
E_PAD = E_PER_TILE * NW
ZCH = 80
N_ZCH = N_NODES // ZCH


def _tc_prologue(user_emb, item_emb, creator_feat, item_feat, Wc, bc, Wi, bi, lw):
    def body(u_ref, i_ref, cf_ref, if_ref, wc_ref, bc_ref, wi_ref, bi_ref,
             lw_ref, emb_ref, acc_ref):
        u = u_ref[...] + jnp.dot(cf_ref[...], wc_ref[...],
                                 preferred_element_type=jnp.float32)
        u = u + bc_ref[...][None, :]
        it = i_ref[...] + jnp.dot(if_ref[...], wi_ref[...],
                                  preferred_element_type=jnp.float32)
        it = it + bi_ref[...][None, :]
        w0 = lw_ref[0]
        emb_ref[pl.ds(0, N_USERS), :] = u
        emb_ref[pl.ds(N_USERS, M_ITEMS), :] = it
        acc_ref[pl.ds(0, N_USERS), :] = u * w0
        acc_ref[pl.ds(N_USERS, M_ITEMS), :] = it * w0

    return pl.pallas_call(
        body,
        out_shape=(
            jax.ShapeDtypeStruct((N_NODES, D), jnp.float32),
            jax.ShapeDtypeStruct((N_NODES, D), jnp.float32),
        ),
        in_specs=[
            pl.BlockSpec(memory_space=pltpu.VMEM),
            pl.BlockSpec(memory_space=pltpu.VMEM),
            pl.BlockSpec(memory_space=pltpu.VMEM),
            pl.BlockSpec(memory_space=pltpu.VMEM),
            pl.BlockSpec(memory_space=pltpu.VMEM),
            pl.BlockSpec(memory_space=pltpu.VMEM),
            pl.BlockSpec(memory_space=pltpu.VMEM),
            pl.BlockSpec(memory_space=pltpu.VMEM),
            pl.BlockSpec(memory_space=pltpu.SMEM),
        ],
        out_specs=(
            pl.BlockSpec(memory_space=pltpu.VMEM),
            pl.BlockSpec(memory_space=pltpu.VMEM),
        ),
    )(user_emb, item_emb, creator_feat, item_feat, Wc, bc, Wi, bi, lw)


def _sc_scatter(emb, rows_p, cols_p, vals_p):
    mesh = plsc.VectorSubcoreMesh(core_axis_name="c", subcore_axis_name="s")

    @functools.partial(
        pl.kernel,
        mesh=mesh,
        out_type=jax.ShapeDtypeStruct((NC, N_NODES, D), jnp.float32),
        scratch_types=[
            pltpu.VMEM_SHARED((N_NODES, D), jnp.float32),
            pltpu.VMEM((CHUNK,), jnp.int32),
            pltpu.VMEM((CHUNK,), jnp.int32),
            pltpu.VMEM((CHUNK,), jnp.float32),
            pltpu.VMEM((CHUNK, D), jnp.float32),
            pltpu.VMEM((ZCH, D), jnp.float32),
            pltpu.SemaphoreType.DMA,
        ],
    )
    def run(emb_hbm, rows_hbm, cols_hbm, vals_hbm, out_hbm,
            acc_sh, colv, roww, valv, gbuf, zbuf, gsem):
        c = lax.axis_index("c")
        s = lax.axis_index("s")

        def zrow(r, _):
            def zcol(d, _):
                zbuf[r, pl.ds(d * 16, 16)] = jnp.zeros((16,), jnp.float32)
                return 0
            return lax.fori_loop(0, D // 16, zcol, 0)
        lax.fori_loop(0, ZCH, zrow, 0)

        def zcopy(k, _):
            idx = s + k * NS

            @pl.when(idx < N_ZCH)
            def _():
                pltpu.sync_copy(zbuf, acc_sh.at[pl.ds(idx * ZCH, ZCH)])
            return 0
        lax.fori_loop(0, (N_ZCH + NS - 1) // NS, zcopy, 0)
        plsc.subcore_barrier()

        base = (c * NS + s) * E_PER_TILE

        def chunk_body(i, _):
            off = base + i * CHUNK
            pltpu.sync_copy(cols_hbm.at[pl.ds(off, CHUNK)], colv)
            pltpu.sync_copy(rows_hbm.at[pl.ds(off, CHUNK)], roww)
            pltpu.sync_copy(vals_hbm.at[pl.ds(off, CHUNK)], valv)
            pltpu.async_copy(emb_hbm.at[colv], gbuf, gsem).wait()

            def scale_group(g, _):
                vvec = valv[pl.ds(g * 16, 16)]

                def scale(j, _):
                    e = g * 16 + j
                    bidx = jnp.broadcast_to(j, (16,)).astype(jnp.int32)
                    dnums = lax.GatherDimensionNumbers(
                        offset_dims=(), collapsed_slice_dims=(0,),
                        start_index_map=(0,))
                    bval = lax.gather(
                        vvec, bidx[:, None], dnums, slice_sizes=(1,),
                        mode=lax.GatherScatterMode.PROMISE_IN_BOUNDS)
                    for dd in range(D // 16):
                        sl = pl.ds(dd * 16, 16)
                        gbuf[e, sl] = gbuf[e, sl] * bval
                    return 0
                return lax.fori_loop(0, 16, scale, 0)
            lax.fori_loop(0, CHUNK // 16, scale_group, 0)

            pltpu.sync_copy(gbuf, acc_sh.at[roww], add=True)
            return 0
        lax.fori_loop(0, N_CHUNKS, chunk_body, 0)
        plsc.subcore_barrier()

        def outcopy(k, _):
            idx = s + k * NS

            @pl.when(idx < N_ZCH)
            def _():
                r0 = idx * ZCH
                pltpu.sync_copy(acc_sh.at[pl.ds(r0, ZCH)], zbuf)
                pltpu.sync_copy(zbuf, out_hbm.at[c, pl.ds(r0, ZCH)])
            return 0
        lax.fori_loop(0, (N_ZCH + NS - 1) // NS, outcopy, 0)

    return run(emb, rows_p, cols_p, vals_p)


CCH = 80
N_CCH = N_NODES // CCH


def _sc_combine(part, acc, wvec16):
    mesh = plsc.VectorSubcoreMesh(core_axis_name="c", subcore_axis_name="s")

    @functools.partial(
        pl.kernel,
        mesh=mesh,
        out_type=(
            jax.ShapeDtypeStruct((N_NODES, D), jnp.float32),
            jax.ShapeDtypeStruct((N_NODES, D), jnp.float32),
        ),
        scratch_types=[
            pltpu.VMEM((CCH, D), jnp.float32),
            pltpu.VMEM((CCH, D), jnp.float32),
            pltpu.VMEM((CCH, D), jnp.float32),
            pltpu.VMEM((16,), jnp.float32),
            pltpu.SemaphoreType.DMA,
        ],
    )
    def run(part_hbm, acc_hbm, w_hbm, emb_out, acc_out, p0v, p1v, av, wv, sem):
        c = lax.axis_index("c")
        s = lax.axis_index("s")
        wid = s * NC + c
        pltpu.sync_copy(w_hbm, wv)
        w = wv[...]

        def do_chunk(k, _):
            idx = wid + k * NW

            @pl.when(idx < N_CCH)
            def _():
                r0 = idx * CCH
                pltpu.sync_copy(part_hbm.at[0, pl.ds(r0, CCH)], p0v)
                pltpu.sync_copy(part_hbm.at[1, pl.ds(r0, CCH)], p1v)
                pltpu.sync_copy(acc_hbm.at[pl.ds(r0, CCH)], av)

                def rbody(r, _):
                    def dbody(d, _):
                        sl = pl.ds(d * 16, 16)
                        ne = p0v[r, sl] + p1v[r, sl]
                        p0v[r, sl] = ne
                        av[r, sl] = av[r, sl] + ne * w
                        return 0
                    return lax.fori_loop(0, D // 16, dbody, 0)
                lax.fori_loop(0, CCH, rbody, 0)

                pltpu.sync_copy(p0v, emb_out.at[pl.ds(r0, CCH)])
                pltpu.sync_copy(av, acc_out.at[pl.ds(r0, CCH)])
            return 0
        lax.fori_loop(0, (N_CCH + NW - 1) // NW, do_chunk, 0)

    return run(part, acc, wvec16)


def kernel(user_emb, item_emb, creator_feat, item_feat, Wc, bc, Wi, bi,
           adj_values, layer_weights, adj_indices):
    rows = adj_indices[0]
    cols = adj_indices[1]
    pad = E_PAD - N_EDGES
    pad_idx = (jnp.arange(pad, dtype=jnp.int32) * 13) % N_NODES
    rows_p = jnp.concatenate([rows, pad_idx])
    cols_p = jnp.concatenate([cols, pad_idx])
    vals_p = jnp.concatenate([adj_values, jnp.zeros((pad,), jnp.float32)])

    emb, acc = _tc_prologue(user_emb, item_emb, creator_feat, item_feat,
                            Wc, bc, Wi, bi, layer_weights)
    for l in range(1, N_LAYERS + 1):
        part = _sc_scatter(emb, rows_p, cols_p, vals_p)
        wvec16 = jnp.broadcast_to(layer_weights[l], (16,))
        emb, acc = _sc_combine(part, acc, wvec16)

    return acc[:N_USERS], acc[N_USERS:]
